# baseline (speedup 1.0000x reference)
import functools

import jax
import jax.numpy as jnp
from jax import lax
from jax.experimental import pallas as pl
from jax.experimental.pallas import tpu as pltpu
from jax.experimental.pallas import tpu_sc as plsc

_NC = 2
_NS = 16
_NW = _NC * _NS


def _mesh():
    return plsc.VectorSubcoreMesh(core_axis_name="c", subcore_axis_name="s")


def _sc_gather(table, idx, ch):
    n, c = table.shape
    e = idx.shape[0]
    per_w = e // _NW
    assert per_w * _NW == e and per_w % ch == 0
    ng = per_w // ch

    @functools.partial(
        pl.kernel,
        out_type=jax.ShapeDtypeStruct((e, c), jnp.float32),
        mesh=_mesh(),
        compiler_params=pltpu.CompilerParams(use_tc_tiling_on_sc=False),
        scratch_types=[
            pltpu.VMEM((ch,), jnp.int32),
            pltpu.VMEM((ch,), jnp.int32),
            pltpu.VMEM((ch, c), jnp.float32),
            pltpu.VMEM((ch, c), jnp.float32),
            pltpu.SemaphoreType.DMA, pltpu.SemaphoreType.DMA,
            pltpu.SemaphoreType.DMA, pltpu.SemaphoreType.DMA,
            pltpu.SemaphoreType.DMA, pltpu.SemaphoreType.DMA,
        ],
    )
    def k(tab_hbm, idx_hbm, out_hbm, ib0, ib1, rb0, rb1,
          is0, is1, gs0, gs1, ss0, ss1):
        cc = lax.axis_index("c")
        ss = lax.axis_index("s")
        base = (ss * _NC + cc) * per_w
        ib = (ib0, ib1)
        rb = (rb0, rb1)
        isem = (is0, is1)
        gsem = (gs0, gs1)
        ssem = (ss0, ss1)

        def load_idx(g, b):
            return pltpu.async_copy(
                idx_hbm.at[pl.ds(base + g * ch, ch)], ib[b], isem[b])

        def start_gather(b):
            return pltpu.async_copy(tab_hbm.at[ib[b]], rb[b], gsem[b])

        def start_store(g, b):
            return pltpu.async_copy(
                rb[b], out_hbm.at[pl.ds(base + g * ch, ch)], ssem[b])

        pend_i = {0: load_idx(0, 0)}
        if ng > 1:
            pend_i[1] = load_idx(1, 1)
        pend_i[0].wait()
        pend_g = {0: start_gather(0)}
        pend_s = {}
        for g in range(ng):
            b = g & 1
            pend_g[g].wait()
            if g + 2 < ng:
                pend_i[g + 2] = load_idx(g + 2, b)
            if g + 1 < ng:
                pend_i[g + 1].wait()
                if g >= 1:
                    pend_s[g - 1].wait()
                pend_g[g + 1] = start_gather(1 - b)
            pend_s[g] = start_store(g, b)
        pend_s[ng - 1].wait()
        if ng > 1:
            pend_s[ng - 2].wait()

    return k(table, idx)


def _sc_degree(dst, ones_hbm, zeros_hbm, n):
    e = dst.shape[0]
    ch = 1000
    per_t = e // _NW
    assert per_t % ch == 0
    ng = per_t // ch
    zch = zeros_hbm.shape[0]
    nz = n // zch
    assert nz * zch == n

    @functools.partial(
        pl.kernel,
        out_type=[jax.ShapeDtypeStruct((n,), jnp.float32),
                  jax.ShapeDtypeStruct((n,), jnp.float32)],
        mesh=_mesh(),
        scratch_types=[
            pltpu.VMEM_SHARED((n,), jnp.float32),
            pltpu.VMEM((ch,), jnp.float32),
            pltpu.VMEM((zch,), jnp.float32),
            pltpu.VMEM((ch,), jnp.int32),
            pltpu.VMEM((ch,), jnp.int32),
            pltpu.SemaphoreType.DMA, pltpu.SemaphoreType.DMA,
        ],
    )
    def k(dst_hbm, ones_h, zero_h, out0_hbm, out1_hbm, acc, ones_v, stage,
          ib0, ib1, is0, is1):
        cc = lax.axis_index("c")
        tt = lax.axis_index("s")
        pltpu.sync_copy(ones_h, ones_v)
        pltpu.sync_copy(zero_h, stage)
        for j in range((nz + _NS - 1) // _NS):
            kk = tt + _NS * j

            @pl.when(kk < nz)
            def _():
                pltpu.sync_copy(stage, acc.at[pl.ds(kk * zch, zch)])
        plsc.subcore_barrier()

        base = cc * (e // _NC) + tt * per_t
        ib = (ib0, ib1)
        isem = (is0, is1)

        def load_idx(g, b):
            return pltpu.async_copy(
                dst_hbm.at[pl.ds(base + g * ch, ch)], ib[b], isem[b])

        pend = {0: load_idx(0, 0)}
        if ng > 1:
            pend[1] = load_idx(1, 1)
        for g in range(ng):
            b = g & 1
            pend[g].wait()
            pltpu.sync_copy(ones_v, acc.at[ib[b]], add=True)
            if g + 2 < ng:
                pend[g + 2] = load_idx(g + 2, b)
        plsc.subcore_barrier()
        for j in range((nz + _NS - 1) // _NS):
            kk = tt + _NS * j

            @pl.when(jnp.logical_and(kk < nz, cc == 0))
            def _():
                pltpu.sync_copy(acc.at[pl.ds(kk * zch, zch)], stage)
                pltpu.sync_copy(stage, out0_hbm.at[pl.ds(kk * zch, zch)])

            @pl.when(jnp.logical_and(kk < nz, cc == 1))
            def _():
                pltpu.sync_copy(acc.at[pl.ds(kk * zch, zch)], stage)
                pltpu.sync_copy(stage, out1_hbm.at[pl.ds(kk * zch, zch)])

    return k(dst, ones_hbm, zeros_hbm)


def _sc_scatter(m, dst, zrows_hbm, n):
    e = dst.shape[0]
    hc = m.shape[2]
    ch = 400
    per_t = e // _NS
    assert per_t % ch == 0
    ng = per_t // ch
    zch = zrows_hbm.shape[0]
    nz = n // zch
    assert nz * zch == n

    @functools.partial(
        pl.kernel,
        out_type=jax.ShapeDtypeStruct((_NC, n, hc), jnp.float32),
        mesh=_mesh(),
        compiler_params=pltpu.CompilerParams(use_tc_tiling_on_sc=False),
        scratch_types=[
            pltpu.VMEM_SHARED((n, hc), jnp.float32),
            pltpu.VMEM((ch,), jnp.int32),
            pltpu.VMEM((ch,), jnp.int32),
            pltpu.VMEM((ch, hc), jnp.float32),
            pltpu.VMEM((ch, hc), jnp.float32),
            pltpu.SemaphoreType.DMA, pltpu.SemaphoreType.DMA,
            pltpu.SemaphoreType.DMA, pltpu.SemaphoreType.DMA,
        ],
    )
    def k(m_hbm, dst_hbm, zrows_h, out_hbm, acc, ib0, ib1, ub0, ub1,
          is0, is1, us0, us1):
        cc = lax.axis_index("c")
        tt = lax.axis_index("s")
        pltpu.sync_copy(zrows_h, ub0)
        for j in range((nz + _NS - 1) // _NS):
            kk = tt + _NS * j

            @pl.when(kk < nz)
            def _():
                pltpu.sync_copy(ub0, acc.at[pl.ds(kk * zch, zch)])
        plsc.subcore_barrier()

        base = tt * per_t
        ib = (ib0, ib1)
        ub = (ub0, ub1)
        isem = (is0, is1)
        usem = (us0, us1)

        def load(g, b):
            di = pltpu.async_copy(
                dst_hbm.at[pl.ds(base + g * ch, ch)], ib[b], isem[b])
            du = pltpu.async_copy(
                m_hbm.at[cc, pl.ds(base + g * ch, ch)], ub[b], usem[b])
            return di, du

        pend = {0: load(0, 0)}
        if ng > 1:
            pend[1] = load(1, 1)
        for g in range(ng):
            b = g & 1
            pend[g][0].wait()
            pend[g][1].wait()
            pltpu.sync_copy(ub[b], acc.at[ib[b]], add=True)
            if g + 2 < ng:
                pend[g + 2] = load(g + 2, b)
        plsc.subcore_barrier()
        for j in range((nz + _NS - 1) // _NS):
            kk = tt + _NS * j

            @pl.when(kk < nz)
            def _():
                pltpu.sync_copy(acc.at[pl.ds(kk * zch, zch)], ub0)
                pltpu.sync_copy(ub0, out_hbm.at[cc, pl.ds(kk * zch, zch)])

    return k(m, dst, zrows_hbm)


def _dot(a, b):
    return jax.lax.dot_general(a, b, (((1,), (0,)), ((), ())),
                               preferred_element_type=jnp.float32)


def _dot0(a, b):
    return jax.lax.dot_general(a, b, (((0,), (0,)), ((), ())),
                               preferred_element_type=jnp.float32)


def _pack(x):
    r, c = x.shape
    k = 128 // c
    x3 = x.reshape(r // k, k, c)
    return jnp.concatenate([x3[:, j] for j in range(k)], axis=1)


def _unpack(p, c):
    q = p.shape[0]
    k = 128 // c
    g3 = jnp.stack([p[:, j * c:(j + 1) * c] for j in range(k)], axis=1)
    return g3.reshape(q * k, c)


def _full(shape):
    return pl.BlockSpec(shape, lambda i: tuple(0 for _ in shape))


def _prep1(xTr, tw1bd, tb1t, tw2bd, tb2t, w1abd, b1t, bnq):
    n4 = xTr.shape[1]

    def body(x_ref, tw1, tb1r, tw2, tb2r, wa, b1r, x0_ref, g1_ref):
        x1 = jnp.maximum(_dot0(x_ref[...], tw1[...]) + tb1r[...], 0.0)
        x0p = jnp.maximum(_dot(x1, tw2[...]) + tb2r[...], 0.0)
        x0_ref[...] = x0p
        g1_ref[...] = _dot(x0p, wa[...]) + b1r[...]

    return pl.pallas_call(
        body,
        grid=(pl.cdiv(n4, bnq),),
        in_specs=[
            pl.BlockSpec((xTr.shape[0], bnq), lambda i: (0, i)),
            _full(tw1bd.shape), _full(tb1t.shape), _full(tw2bd.shape),
            _full(tb2t.shape), _full(w1abd.shape), _full(b1t.shape),
        ],
        out_specs=[
            pl.BlockSpec((bnq, 128), lambda i: (i, 0)),
            pl.BlockSpec((bnq, 128), lambda i: (i, 0)),
        ],
        out_shape=[
            jax.ShapeDtypeStruct((n4, 128), jnp.float32),
            jax.ShapeDtypeStruct((n4, 128), jnp.float32),
        ],
    )(xTr, tw1bd, tb1t, tw2bd, tb2t, w1abd, b1t)


def _edge_mlp(garrp, eaR, w1big, w2lo, w2hi, b2lo, b2hi, aub):
    e4, gw = garrp.shape

    def body(g_ref, ea_ref, w1, wlo, whi, blo, bhi, out_ref):
        et = _dot0(ea_ref[...], w1[...])
        h = jnp.maximum(g_ref[...] + et, 0.0)
        out_ref[0] = jnp.maximum(_dot(h, wlo[...]) + blo[...], 0.0)
        out_ref[1] = jnp.maximum(_dot(h, whi[...]) + bhi[...], 0.0)

    return pl.pallas_call(
        body,
        grid=(pl.cdiv(e4, aub),),
        in_specs=[
            pl.BlockSpec((aub, gw), lambda i: (i, 0)),
            pl.BlockSpec((64, aub), lambda i: (0, i)),
            _full(w1big.shape), _full(w2lo.shape), _full(w2hi.shape),
            _full(b2lo.shape), _full(b2hi.shape),
        ],
        out_specs=pl.BlockSpec((2, aub, 128), lambda i: (0, i, 0)),
        out_shape=jax.ShapeDtypeStruct((2, e4, 128), jnp.float32),
    )(garrp, eaR, w1big, w2lo, w2hi, b2lo, b2hi)


def _mid(aggp, d4, x0p, bdlo, bdhi, bdx, b1t, m256, bnq):
    n4 = x0p.shape[0]

    def body(a_ref, d_ref, x0_ref, wl, wh, wxr, b1r, mr, out_ref):
        inv4 = 1.0 / jnp.maximum(d_ref[...], 1.0)
        invp = _dot(inv4, mr[...])
        a = a_ref[...]
        g2 = (invp * (_dot(a[0], wl[...]) + _dot(a[1], wh[...]))
              + _dot(x0_ref[...], wxr[...]) + b1r[...])
        out_ref[...] = g2

    return pl.pallas_call(
        body,
        grid=(pl.cdiv(n4, bnq),),
        in_specs=[
            pl.BlockSpec((2, bnq, 128), lambda i: (0, i, 0)),
            pl.BlockSpec((bnq, 4), lambda i: (i, 0)),
            pl.BlockSpec((bnq, 128), lambda i: (i, 0)),
            _full(bdlo.shape), _full(bdhi.shape), _full(bdx.shape),
            _full(b1t.shape), _full(m256.shape),
        ],
        out_specs=pl.BlockSpec((bnq, 256), lambda i: (i, 0)),
        out_shape=jax.ShapeDtypeStruct((n4, 256), jnp.float32),
    )(aggp, d4, x0p, bdlo, bdhi, bdx, b1t, m256)


def _head(aggp, d4, x0p, bdlo, bdhi, bdx, pb1t, bdw2, pb2t, m64, bnq):
    n4 = x0p.shape[0]

    def body(a_ref, d_ref, x0_ref, wl, wh, wxr, b1r, w2r, b2r, mr, out_ref):
        inv4 = 1.0 / jnp.maximum(d_ref[...], 1.0)
        invp = _dot(inv4, mr[...])
        a = a_ref[...]
        h = jnp.maximum(
            invp * (_dot(a[0], wl[...]) + _dot(a[1], wh[...]))
            + _dot(x0_ref[...], wxr[...]) + b1r[...], 0.0)
        out_ref[...] = _dot(h, w2r[...]) + b2r[...]

    return pl.pallas_call(
        body,
        grid=(pl.cdiv(n4, bnq),),
        in_specs=[
            pl.BlockSpec((2, bnq, 128), lambda i: (0, i, 0)),
            pl.BlockSpec((bnq, 4), lambda i: (i, 0)),
            pl.BlockSpec((bnq, 128), lambda i: (i, 0)),
            _full(bdlo.shape), _full(bdhi.shape), _full(bdx.shape),
            _full(pb1t.shape), _full(bdw2.shape), _full(pb2t.shape),
            _full(m64.shape),
        ],
        out_specs=pl.BlockSpec((bnq, 4), lambda i: (i, 0)),
        out_shape=jax.ShapeDtypeStruct((n4, 4), jnp.float32),
    )(aggp, d4, x0p, bdlo, bdhi, bdx, pb1t, bdw2, pb2t, m64)


def kernel(x, sx, edge_index, edge_attr, batch, tW1, tb1, tW2, tb2,
           s1W1, s1b1, s1W2, s1b2, s2W1, s2b1, s2W2, s2b2,
           pW1, pb1, pW2, pb2):
    n = x.shape[0]
    e = edge_attr.shape[0]
    n4 = n // 4
    e4 = e // 4
    src = edge_index[0]
    dst = edge_index[1]

    bnq = 512
    aub = 1024

    ones1 = jnp.ones((1000,), jnp.float32)
    z1 = jnp.zeros((2000,), jnp.float32)
    z32 = jnp.zeros((400, 32), jnp.float32)

    i4 = jnp.eye(4, dtype=jnp.float32)

    def bdl(w):
        return (w[:, None, None, :] * i4[None, :, :, None]).reshape(
            4 * w.shape[0], 4 * w.shape[1])

    def bdr(w):
        return (i4[:, None, :, None] * w[None, :, None, :]).reshape(
            4 * w.shape[0], 4 * w.shape[1])

    def t4(b):
        return jnp.tile(b, 4).reshape(1, -1)

    m256 = bdr(jnp.ones((1, 64), jnp.float32))
    m64 = bdr(jnp.ones((1, 16), jnp.float32))

    tsrc = 4 * (src % n4) + src // n4
    tdst = 4 * (dst % n4) + dst // n4
    srcp = tsrc.reshape(4, e4).T.reshape(e)
    dstp = tdst.reshape(4, e4).T.reshape(e)

    degp0, degp1 = _sc_degree(tdst, ones1, z1, n)
    d4 = (degp0 + degp1).reshape(n4, 4)

    eaR = edge_attr.T.reshape(64, e4)
    xTr = x.T.reshape(448, n4)

    x0p, g1p = _prep1(xTr, bdl(tW1), t4(tb1), bdr(tW2), t4(tb2),
                      bdr(s1W1[:32]), t4(s1b1), bnq)

    garr1 = _sc_gather(g1p.reshape(n, 32), srcp, 1000)
    m1 = _edge_mlp(garr1.reshape(e4, 128), eaR, bdl(s1W1[32:48]),
                   bdr(s1W2[:, :32]), bdr(s1W2[:, 32:]),
                   t4(s1b2[:32]), t4(s1b2[32:]), aub)
    agg1 = _sc_scatter(m1.reshape(2, e, 32), dstp, z32, n)

    g2p = _mid(agg1.reshape(2, n4, 128), d4, x0p, bdr(s2W1[:32]),
               bdr(s2W1[32:64]), bdr(s2W1[64:96]), t4(s2b1), m256, bnq)

    garr2 = _sc_gather(g2p.reshape(n, 64), srcp, 1000)
    m2 = _edge_mlp(garr2.reshape(e4, 256), eaR, bdl(s2W1[96:112]),
                   bdr(s2W2[:, :32]), bdr(s2W2[:, 32:]),
                   t4(s2b2[:32]), t4(s2b2[32:]), aub)
    agg2 = _sc_scatter(m2.reshape(2, e, 32), dstp, z32, n)

    y4 = _head(agg2.reshape(2, n4, 128), d4, x0p, bdr(pW1[:32]),
               bdr(pW1[32:64]), bdr(pW1[64:96]), t4(pb1), bdr(pW2),
               t4(pb2), m64, bnq)
    return y4.T.reshape(n, 1)


# speedup vs baseline: 1.9541x; 1.9541x over previous
import functools

import jax
import jax.numpy as jnp
from jax import lax
from jax.experimental import pallas as pl
from jax.experimental.pallas import tpu as pltpu
from jax.experimental.pallas import tpu_sc as plsc

_NC = 2
_NS = 16
_NW = _NC * _NS


def _mesh():
    return plsc.VectorSubcoreMesh(core_axis_name="c", subcore_axis_name="s")


def _sc_gather(table, idx, ch):
    n, c = table.shape
    e = idx.shape[0]
    per_w = e // _NW
    assert per_w * _NW == e and per_w % ch == 0
    ng = per_w // ch

    @functools.partial(
        pl.kernel,
        out_type=jax.ShapeDtypeStruct((e, c), jnp.float32),
        mesh=_mesh(),
        compiler_params=pltpu.CompilerParams(use_tc_tiling_on_sc=False),
        scratch_types=[
            pltpu.VMEM((ch,), jnp.int32),
            pltpu.VMEM((ch,), jnp.int32),
            pltpu.VMEM((ch, c), jnp.float32),
            pltpu.VMEM((ch, c), jnp.float32),
            pltpu.SemaphoreType.DMA, pltpu.SemaphoreType.DMA,
            pltpu.SemaphoreType.DMA, pltpu.SemaphoreType.DMA,
            pltpu.SemaphoreType.DMA, pltpu.SemaphoreType.DMA,
        ],
    )
    def k(tab_hbm, idx_hbm, out_hbm, ib0, ib1, rb0, rb1,
          is0, is1, gs0, gs1, ss0, ss1):
        cc = lax.axis_index("c")
        ss = lax.axis_index("s")
        base = (ss * _NC + cc) * per_w
        ib = (ib0, ib1)
        rb = (rb0, rb1)
        isem = (is0, is1)
        gsem = (gs0, gs1)
        ssem = (ss0, ss1)

        def load_idx(g, b):
            return pltpu.async_copy(
                idx_hbm.at[pl.ds(base + g * ch, ch)], ib[b], isem[b])

        def start_gather(b):
            return pltpu.async_copy(tab_hbm.at[ib[b]], rb[b], gsem[b])

        def start_store(g, b):
            return pltpu.async_copy(
                rb[b], out_hbm.at[pl.ds(base + g * ch, ch)], ssem[b])

        pend_i = {0: load_idx(0, 0)}
        if ng > 1:
            pend_i[1] = load_idx(1, 1)
        pend_i[0].wait()
        pend_g = {0: start_gather(0)}
        pend_s = {}
        for g in range(ng):
            b = g & 1
            pend_g[g].wait()
            if g + 2 < ng:
                pend_i[g + 2] = load_idx(g + 2, b)
            if g + 1 < ng:
                pend_i[g + 1].wait()
                if g >= 1:
                    pend_s[g - 1].wait()
                pend_g[g + 1] = start_gather(1 - b)
            pend_s[g] = start_store(g, b)
        pend_s[ng - 1].wait()
        if ng > 1:
            pend_s[ng - 2].wait()

    return k(table, idx)


def _sc_index_prep(src, dst, pattern_hbm, ones_hbm, zeros_hbm, n):
    e = src.shape[0]
    n4 = n // 4
    e4 = e // 4
    ch = pattern_hbm.shape[0]
    run = ch // 4
    nchunks = e // ch
    nt = (nchunks + _NW - 1) // _NW
    zch = zeros_hbm.shape[0]
    nz = n // zch
    assert nz * zch == n and nchunks * ch == e and (e4 % 8 == 0)

    @functools.partial(
        pl.kernel,
        out_type=[jax.ShapeDtypeStruct((e,), jnp.int32),
                  jax.ShapeDtypeStruct((e,), jnp.int32),
                  jax.ShapeDtypeStruct((n,), jnp.float32),
                  jax.ShapeDtypeStruct((n,), jnp.float32)],
        mesh=_mesh(),
        compiler_params=pltpu.CompilerParams(needs_layout_passes=False),
        scratch_types=[
            pltpu.VMEM_SHARED((n,), jnp.float32),
            pltpu.VMEM((ch,), jnp.int32),
            pltpu.VMEM((ch,), jnp.int32),
            pltpu.VMEM((ch,), jnp.int32),
            pltpu.VMEM((ch,), jnp.int32),
            pltpu.VMEM((ch,), jnp.int32),
            pltpu.VMEM((ch,), jnp.float32),
            pltpu.VMEM((zch,), jnp.float32),
            pltpu.SemaphoreType.DMA,
        ],
    )
    def k(src_h, dst_h, pat_h, ones_h, zero_h, srcp_h, dstp_h, d0_h, d1_h,
          acc, sbuf, dbuf, osb, odb, patv, onesv, stage, sem):
        cc = lax.axis_index("c")
        tt = lax.axis_index("s")
        wid = tt * _NC + cc
        pltpu.sync_copy(pat_h, patv)
        pltpu.sync_copy(ones_h, onesv)
        pltpu.sync_copy(zero_h, stage)
        for j in range((nz + _NS - 1) // _NS):
            kk = tt + _NS * j

            @pl.when(kk < nz)
            def _():
                pltpu.sync_copy(stage, acc.at[pl.ds(kk * zch, zch)])
        plsc.subcore_barrier()

        for t in range(nt):
            k_ = wid + _NW * t

            @pl.when(k_ < nchunks)
            def _():
                r0 = k_ * run
                pend = []
                for j in range(4):
                    pend.append(pltpu.async_copy(
                        src_h.at[pl.ds(j * e4 + r0, run)],
                        sbuf.at[pl.ds(j * run, run)], sem))
                    pend.append(pltpu.async_copy(
                        dst_h.at[pl.ds(j * e4 + r0, run)],
                        dbuf.at[pl.ds(j * run, run)], sem))
                for d in pend:
                    d.wait()

                @pl.loop(0, ch, step=16)
                def _(i):
                    pi = patv[pl.ds(i, 16)]
                    for inb, outb in ((sbuf, osb), (dbuf, odb)):
                        v = inb[pl.ds(i, 16)]
                        jj = ((v >= n4).astype(jnp.int32)
                              + (v >= 2 * n4).astype(jnp.int32)
                              + (v >= 3 * n4).astype(jnp.int32))
                        tv = 4 * v - jj * (4 * n4 - 1)
                        plsc.store_scatter(outb, [pi], tv)

                pltpu.sync_copy(onesv, acc.at[odb], add=True)
                pltpu.sync_copy(osb, srcp_h.at[pl.ds(k_ * ch, ch)])
                pltpu.sync_copy(odb, dstp_h.at[pl.ds(k_ * ch, ch)])
        plsc.subcore_barrier()
        for j in range((nz + _NS - 1) // _NS):
            kk = tt + _NS * j

            @pl.when(jnp.logical_and(kk < nz, cc == 0))
            def _():
                pltpu.sync_copy(acc.at[pl.ds(kk * zch, zch)], stage)
                pltpu.sync_copy(stage, d0_h.at[pl.ds(kk * zch, zch)])

            @pl.when(jnp.logical_and(kk < nz, cc == 1))
            def _():
                pltpu.sync_copy(acc.at[pl.ds(kk * zch, zch)], stage)
                pltpu.sync_copy(stage, d1_h.at[pl.ds(kk * zch, zch)])

    return k(src, dst, pattern_hbm, ones_hbm, zeros_hbm)


def _sc_scatter(m, dst, zrows_hbm, n):
    e = dst.shape[0]
    hc = m.shape[2]
    ch = 400
    per_t = e // _NS
    assert per_t % ch == 0
    ng = per_t // ch
    zch = zrows_hbm.shape[0]
    nz = n // zch
    assert nz * zch == n

    @functools.partial(
        pl.kernel,
        out_type=jax.ShapeDtypeStruct((_NC, n, hc), jnp.float32),
        mesh=_mesh(),
        compiler_params=pltpu.CompilerParams(use_tc_tiling_on_sc=False),
        scratch_types=[
            pltpu.VMEM_SHARED((n, hc), jnp.float32),
            pltpu.VMEM((ch,), jnp.int32),
            pltpu.VMEM((ch,), jnp.int32),
            pltpu.VMEM((ch, hc), jnp.float32),
            pltpu.VMEM((ch, hc), jnp.float32),
            pltpu.SemaphoreType.DMA, pltpu.SemaphoreType.DMA,
            pltpu.SemaphoreType.DMA, pltpu.SemaphoreType.DMA,
        ],
    )
    def k(m_hbm, dst_hbm, zrows_h, out_hbm, acc, ib0, ib1, ub0, ub1,
          is0, is1, us0, us1):
        cc = lax.axis_index("c")
        tt = lax.axis_index("s")
        pltpu.sync_copy(zrows_h, ub0)
        for j in range((nz + _NS - 1) // _NS):
            kk = tt + _NS * j

            @pl.when(kk < nz)
            def _():
                pltpu.sync_copy(ub0, acc.at[pl.ds(kk * zch, zch)])
        plsc.subcore_barrier()

        base = tt * per_t
        ib = (ib0, ib1)
        ub = (ub0, ub1)
        isem = (is0, is1)
        usem = (us0, us1)

        def load(g, b):
            di = pltpu.async_copy(
                dst_hbm.at[pl.ds(base + g * ch, ch)], ib[b], isem[b])
            du = pltpu.async_copy(
                m_hbm.at[cc, pl.ds(base + g * ch, ch)], ub[b], usem[b])
            return di, du

        pend = {0: load(0, 0)}
        if ng > 1:
            pend[1] = load(1, 1)
        for g in range(ng):
            b = g & 1
            pend[g][0].wait()
            pend[g][1].wait()
            pltpu.sync_copy(ub[b], acc.at[ib[b]], add=True)
            if g + 2 < ng:
                pend[g + 2] = load(g + 2, b)
        plsc.subcore_barrier()
        for j in range((nz + _NS - 1) // _NS):
            kk = tt + _NS * j

            @pl.when(kk < nz)
            def _():
                pltpu.sync_copy(acc.at[pl.ds(kk * zch, zch)], ub0)
                pltpu.sync_copy(ub0, out_hbm.at[cc, pl.ds(kk * zch, zch)])

    return k(m, dst, zrows_hbm)


def _dot(a, b):
    return jax.lax.dot_general(a, b, (((1,), (0,)), ((), ())),
                               preferred_element_type=jnp.float32)


def _dot0(a, b):
    return jax.lax.dot_general(a, b, (((0,), (0,)), ((), ())),
                               preferred_element_type=jnp.float32)


def _pack(x):
    r, c = x.shape
    k = 128 // c
    x3 = x.reshape(r // k, k, c)
    return jnp.concatenate([x3[:, j] for j in range(k)], axis=1)


def _unpack(p, c):
    q = p.shape[0]
    k = 128 // c
    g3 = jnp.stack([p[:, j * c:(j + 1) * c] for j in range(k)], axis=1)
    return g3.reshape(q * k, c)


def _full(shape):
    return pl.BlockSpec(shape, lambda i: tuple(0 for _ in shape))


def _prep1(xTr, tw1bd, tb1t, tw2bd, tb2t, w1abd, b1t, bnq):
    n4 = xTr.shape[1]

    def body(x_ref, tw1, tb1r, tw2, tb2r, wa, b1r, x0_ref, g1_ref):
        x1 = jnp.maximum(_dot0(x_ref[...], tw1[...]) + tb1r[...], 0.0)
        x0p = jnp.maximum(_dot(x1, tw2[...]) + tb2r[...], 0.0)
        x0_ref[...] = x0p
        g1_ref[...] = _dot(x0p, wa[...]) + b1r[...]

    return pl.pallas_call(
        body,
        grid=(pl.cdiv(n4, bnq),),
        in_specs=[
            pl.BlockSpec((xTr.shape[0], bnq), lambda i: (0, i)),
            _full(tw1bd.shape), _full(tb1t.shape), _full(tw2bd.shape),
            _full(tb2t.shape), _full(w1abd.shape), _full(b1t.shape),
        ],
        out_specs=[
            pl.BlockSpec((bnq, 128), lambda i: (i, 0)),
            pl.BlockSpec((bnq, 128), lambda i: (i, 0)),
        ],
        out_shape=[
            jax.ShapeDtypeStruct((n4, 128), jnp.float32),
            jax.ShapeDtypeStruct((n4, 128), jnp.float32),
        ],
    )(xTr, tw1bd, tb1t, tw2bd, tb2t, w1abd, b1t)


def _edge_mlp(garrp, eaR, w1big, w2lo, w2hi, b2lo, b2hi, aub):
    e4, gw = garrp.shape

    def body(g_ref, ea_ref, w1, wlo, whi, blo, bhi, out_ref):
        et = _dot0(ea_ref[...], w1[...])
        h = jnp.maximum(g_ref[...] + et, 0.0)
        out_ref[0] = jnp.maximum(_dot(h, wlo[...]) + blo[...], 0.0)
        out_ref[1] = jnp.maximum(_dot(h, whi[...]) + bhi[...], 0.0)

    return pl.pallas_call(
        body,
        grid=(pl.cdiv(e4, aub),),
        in_specs=[
            pl.BlockSpec((aub, gw), lambda i: (i, 0)),
            pl.BlockSpec((64, aub), lambda i: (0, i)),
            _full(w1big.shape), _full(w2lo.shape), _full(w2hi.shape),
            _full(b2lo.shape), _full(b2hi.shape),
        ],
        out_specs=pl.BlockSpec((2, aub, 128), lambda i: (0, i, 0)),
        out_shape=jax.ShapeDtypeStruct((2, e4, 128), jnp.float32),
    )(garrp, eaR, w1big, w2lo, w2hi, b2lo, b2hi)


def _edge_mlp2(glo, ghi, eaR, w1lo, w1hi, w2ll, w2hl, w2lh, w2hh,
               b2lo, b2hi, aub):
    e4 = glo.shape[0]

    def body(gl_ref, gh_ref, ea_ref, wlo, whi, ll, hl, lh, hh, blo, bhi,
             out_ref):
        ea = ea_ref[...]
        h_lo = jnp.maximum(gl_ref[...] + _dot0(ea, wlo[...]), 0.0)
        h_hi = jnp.maximum(gh_ref[...] + _dot0(ea, whi[...]), 0.0)
        out_ref[0] = jnp.maximum(
            _dot(h_lo, ll[...]) + _dot(h_hi, hl[...]) + blo[...], 0.0)
        out_ref[1] = jnp.maximum(
            _dot(h_lo, lh[...]) + _dot(h_hi, hh[...]) + bhi[...], 0.0)

    return pl.pallas_call(
        body,
        grid=(pl.cdiv(e4, aub),),
        in_specs=[
            pl.BlockSpec((aub, 128), lambda i: (i, 0)),
            pl.BlockSpec((aub, 128), lambda i: (i, 0)),
            pl.BlockSpec((64, aub), lambda i: (0, i)),
            _full(w1lo.shape), _full(w1hi.shape), _full(w2ll.shape),
            _full(w2hl.shape), _full(w2lh.shape), _full(w2hh.shape),
            _full(b2lo.shape), _full(b2hi.shape),
        ],
        out_specs=pl.BlockSpec((2, aub, 128), lambda i: (0, i, 0)),
        out_shape=jax.ShapeDtypeStruct((2, e4, 128), jnp.float32),
    )(glo, ghi, eaR, w1lo, w1hi, w2ll, w2hl, w2lh, w2hh, b2lo, b2hi)


def _mid(aggp, d4, x0p, bdlos, bdhis, bdxs, b1ts, m128, bnq):
    n4 = x0p.shape[0]

    def body(a_ref, d_ref, x0_ref, wl0, wl1, wh0, wh1, wx0, wx1, b0, b1r,
             mr, lo_ref, hi_ref):
        inv4 = 1.0 / jnp.maximum(d_ref[...], 1.0)
        invp = _dot(inv4, mr[...])
        a = a_ref[...]
        lo_ref[...] = (invp * (_dot(a[0], wl0[...]) + _dot(a[1], wh0[...]))
                       + _dot(x0_ref[...], wx0[...]) + b0[...])
        hi_ref[...] = (invp * (_dot(a[0], wl1[...]) + _dot(a[1], wh1[...]))
                       + _dot(x0_ref[...], wx1[...]) + b1r[...])

    return pl.pallas_call(
        body,
        grid=(pl.cdiv(n4, bnq),),
        in_specs=[
            pl.BlockSpec((2, bnq, 128), lambda i: (0, i, 0)),
            pl.BlockSpec((bnq, 4), lambda i: (i, 0)),
            pl.BlockSpec((bnq, 128), lambda i: (i, 0)),
            _full(bdlos[0].shape), _full(bdlos[1].shape),
            _full(bdhis[0].shape), _full(bdhis[1].shape),
            _full(bdxs[0].shape), _full(bdxs[1].shape),
            _full(b1ts[0].shape), _full(b1ts[1].shape),
            _full(m128.shape),
        ],
        out_specs=[
            pl.BlockSpec((bnq, 128), lambda i: (i, 0)),
            pl.BlockSpec((bnq, 128), lambda i: (i, 0)),
        ],
        out_shape=[
            jax.ShapeDtypeStruct((n4, 128), jnp.float32),
            jax.ShapeDtypeStruct((n4, 128), jnp.float32),
        ],
    )(aggp, d4, x0p, bdlos[0], bdlos[1], bdhis[0], bdhis[1],
      bdxs[0], bdxs[1], b1ts[0], b1ts[1], m128)


def _head(aggp, d4, x0p, bdlo, bdhi, bdx, pb1t, bdw2, pb2t, m64, bnq):
    n4 = x0p.shape[0]

    def body(a_ref, d_ref, x0_ref, wl, wh, wxr, b1r, w2r, b2r, mr, out_ref):
        inv4 = 1.0 / jnp.maximum(d_ref[...], 1.0)
        invp = _dot(inv4, mr[...])
        a = a_ref[...]
        h = jnp.maximum(
            invp * (_dot(a[0], wl[...]) + _dot(a[1], wh[...]))
            + _dot(x0_ref[...], wxr[...]) + b1r[...], 0.0)
        out_ref[...] = _dot(h, w2r[...]) + b2r[...]

    return pl.pallas_call(
        body,
        grid=(pl.cdiv(n4, bnq),),
        in_specs=[
            pl.BlockSpec((2, bnq, 128), lambda i: (0, i, 0)),
            pl.BlockSpec((bnq, 4), lambda i: (i, 0)),
            pl.BlockSpec((bnq, 128), lambda i: (i, 0)),
            _full(bdlo.shape), _full(bdhi.shape), _full(bdx.shape),
            _full(pb1t.shape), _full(bdw2.shape), _full(pb2t.shape),
            _full(m64.shape),
        ],
        out_specs=pl.BlockSpec((bnq, 4), lambda i: (i, 0)),
        out_shape=jax.ShapeDtypeStruct((n4, 4), jnp.float32),
    )(aggp, d4, x0p, bdlo, bdhi, bdx, pb1t, bdw2, pb2t, m64)


def kernel(x, sx, edge_index, edge_attr, batch, tW1, tb1, tW2, tb2,
           s1W1, s1b1, s1W2, s1b2, s2W1, s2b1, s2W2, s2b2,
           pW1, pb1, pW2, pb2):
    n = x.shape[0]
    e = edge_attr.shape[0]
    n4 = n // 4
    e4 = e // 4
    src = edge_index[0]
    dst = edge_index[1]

    bnq = 512
    aub = 2048
    pch = 4000

    z1 = jnp.zeros((2000,), jnp.float32)
    z32 = jnp.zeros((400, 32), jnp.float32)
    onesp = jnp.ones((pch,), jnp.float32)
    ar = jnp.arange(pch, dtype=jnp.int32)
    pattern = 4 * (ar % (pch // 4)) + ar // (pch // 4)

    i4 = jnp.eye(4, dtype=jnp.float32)

    def bdl(w):
        return (w[:, None, None, :] * i4[None, :, :, None]).reshape(
            4 * w.shape[0], 4 * w.shape[1])

    def bdr(w):
        return (i4[:, None, :, None] * w[None, :, None, :]).reshape(
            4 * w.shape[0], 4 * w.shape[1])

    def t4(b):
        return jnp.tile(b, 4).reshape(1, -1)

    m128 = bdr(jnp.ones((1, 32), jnp.float32))
    m64 = bdr(jnp.ones((1, 16), jnp.float32))

    srcp, dstp, degp0, degp1 = _sc_index_prep(src, dst, pattern, onesp,
                                              z1, n)
    d4 = (degp0 + degp1).reshape(n4, 4)

    eaR = edge_attr.T.reshape(64, e4)
    xTr = x.T.reshape(448, n4)

    x0p, g1p = _prep1(xTr, bdl(tW1), t4(tb1), bdr(tW2), t4(tb2),
                      bdr(s1W1[:32]), t4(s1b1), bnq)

    garr1 = _sc_gather(g1p.reshape(n, 32), srcp, 1000)
    m1 = _edge_mlp(garr1.reshape(e4, 128), eaR, bdl(s1W1[32:48]),
                   bdr(s1W2[:, :32]), bdr(s1W2[:, 32:]),
                   t4(s1b2[:32]), t4(s1b2[32:]), aub)
    agg1 = _sc_scatter(m1.reshape(2, e, 32), dstp, z32, n)

    g2lo, g2hi = _mid(
        agg1.reshape(2, n4, 128), d4, x0p,
        (bdr(s2W1[:32, :32]), bdr(s2W1[:32, 32:])),
        (bdr(s2W1[32:64, :32]), bdr(s2W1[32:64, 32:])),
        (bdr(s2W1[64:96, :32]), bdr(s2W1[64:96, 32:])),
        (t4(s2b1[:32]), t4(s2b1[32:])), m128, bnq)

    glo = _sc_gather(g2lo.reshape(n, 32), srcp, 1000)
    ghi = _sc_gather(g2hi.reshape(n, 32), srcp, 1000)
    m2 = _edge_mlp2(glo.reshape(e4, 128), ghi.reshape(e4, 128), eaR,
                    bdl(s2W1[96:112, :32]), bdl(s2W1[96:112, 32:]),
                    bdr(s2W2[:32, :32]), bdr(s2W2[32:, :32]),
                    bdr(s2W2[:32, 32:]), bdr(s2W2[32:, 32:]),
                    t4(s2b2[:32]), t4(s2b2[32:]), aub)
    agg2 = _sc_scatter(m2.reshape(2, e, 32), dstp, z32, n)

    y4 = _head(agg2.reshape(2, n4, 128), d4, x0p, bdr(pW1[:32]),
               bdr(pW1[32:64]), bdr(pW1[64:96]), t4(pb1), bdr(pW2),
               t4(pb2), m64, bnq)
    return y4.T.reshape(n, 1)


# speedup vs baseline: 2.1173x; 1.0835x over previous
import functools

import jax
import jax.numpy as jnp
from jax import lax
from jax.experimental import pallas as pl
from jax.experimental.pallas import tpu as pltpu
from jax.experimental.pallas import tpu_sc as plsc

_NC = 2
_NS = 16
_NW = _NC * _NS


def _mesh():
    return plsc.VectorSubcoreMesh(core_axis_name="c", subcore_axis_name="s")


def _sc_gather(table, idx, ch):
    n, c = table.shape
    e = idx.shape[0]
    per_w = e // _NW
    assert per_w * _NW == e and per_w % ch == 0
    ng = per_w // ch

    @functools.partial(
        pl.kernel,
        out_type=jax.ShapeDtypeStruct((e, c), jnp.float32),
        mesh=_mesh(),
        compiler_params=pltpu.CompilerParams(use_tc_tiling_on_sc=False),
        scratch_types=[
            pltpu.VMEM((ch,), jnp.int32),
            pltpu.VMEM((ch,), jnp.int32),
            pltpu.VMEM((ch, c), jnp.float32),
            pltpu.VMEM((ch, c), jnp.float32),
            pltpu.SemaphoreType.DMA, pltpu.SemaphoreType.DMA,
            pltpu.SemaphoreType.DMA, pltpu.SemaphoreType.DMA,
            pltpu.SemaphoreType.DMA, pltpu.SemaphoreType.DMA,
        ],
    )
    def k(tab_hbm, idx_hbm, out_hbm, ib0, ib1, rb0, rb1,
          is0, is1, gs0, gs1, ss0, ss1):
        cc = lax.axis_index("c")
        ss = lax.axis_index("s")
        base = (ss * _NC + cc) * per_w
        ib = (ib0, ib1)
        rb = (rb0, rb1)
        isem = (is0, is1)
        gsem = (gs0, gs1)
        ssem = (ss0, ss1)

        def load_idx(g, b):
            return pltpu.async_copy(
                idx_hbm.at[pl.ds(base + g * ch, ch)], ib[b], isem[b])

        def start_gather(b):
            return pltpu.async_copy(tab_hbm.at[ib[b]], rb[b], gsem[b])

        def start_store(g, b):
            return pltpu.async_copy(
                rb[b], out_hbm.at[pl.ds(base + g * ch, ch)], ssem[b])

        pend_i = {0: load_idx(0, 0)}
        if ng > 1:
            pend_i[1] = load_idx(1, 1)
        pend_i[0].wait()
        pend_g = {0: start_gather(0)}
        pend_s = {}
        for g in range(ng):
            b = g & 1
            pend_g[g].wait()
            if g + 2 < ng:
                pend_i[g + 2] = load_idx(g + 2, b)
            if g + 1 < ng:
                pend_i[g + 1].wait()
                if g >= 1:
                    pend_s[g - 1].wait()
                pend_g[g + 1] = start_gather(1 - b)
            pend_s[g] = start_store(g, b)
        pend_s[ng - 1].wait()
        if ng > 1:
            pend_s[ng - 2].wait()

    return k(table, idx)


def _sc_index_prep(ei1, pattern_hbm, ones_hbm, zeros_hbm, n):
    e = ei1.shape[0] // 2
    n4 = n // 4
    e4 = e // 4
    ch = pattern_hbm.shape[0]
    run = ch // 4
    nchunks = e // ch
    nt = (nchunks + _NW - 1) // _NW
    zch = zeros_hbm.shape[0]
    nz = n // zch
    assert nz * zch == n and nchunks * ch == e and (e4 % 8 == 0)

    @functools.partial(
        pl.kernel,
        out_type=[jax.ShapeDtypeStruct((e,), jnp.int32),
                  jax.ShapeDtypeStruct((e,), jnp.int32),
                  jax.ShapeDtypeStruct((n,), jnp.float32),
                  jax.ShapeDtypeStruct((n,), jnp.float32)],
        mesh=_mesh(),
        compiler_params=pltpu.CompilerParams(needs_layout_passes=False),
        scratch_types=[
            pltpu.VMEM_SHARED((n,), jnp.float32),
            pltpu.VMEM((ch,), jnp.int32),
            pltpu.VMEM((ch,), jnp.int32),
            pltpu.VMEM((ch,), jnp.int32),
            pltpu.VMEM((ch,), jnp.int32),
            pltpu.VMEM((ch,), jnp.int32),
            pltpu.VMEM((ch,), jnp.float32),
            pltpu.VMEM((zch,), jnp.float32),
            pltpu.SemaphoreType.DMA,
        ],
    )
    def k(ei_h, pat_h, ones_h, zero_h, srcp_h, dstp_h, d0_h, d1_h,
          acc, sbuf, dbuf, osb, odb, patv, onesv, stage, sem):
        cc = lax.axis_index("c")
        tt = lax.axis_index("s")
        wid = tt * _NC + cc
        pltpu.sync_copy(pat_h, patv)
        pltpu.sync_copy(ones_h, onesv)
        pltpu.sync_copy(zero_h, stage)
        for j in range((nz + _NS - 1) // _NS):
            kk = tt + _NS * j

            @pl.when(kk < nz)
            def _():
                pltpu.sync_copy(stage, acc.at[pl.ds(kk * zch, zch)])
        plsc.subcore_barrier()

        for t in range(nt):
            k_ = wid + _NW * t

            @pl.when(k_ < nchunks)
            def _():
                r0 = k_ * run
                pend = []
                for j in range(4):
                    pend.append(pltpu.async_copy(
                        ei_h.at[pl.ds(j * e4 + r0, run)],
                        sbuf.at[pl.ds(j * run, run)], sem))
                    pend.append(pltpu.async_copy(
                        ei_h.at[pl.ds(e + j * e4 + r0, run)],
                        dbuf.at[pl.ds(j * run, run)], sem))
                for d in pend:
                    d.wait()

                @pl.loop(0, ch, step=16)
                def _(i):
                    pi = patv[pl.ds(i, 16)]
                    for inb, outb in ((sbuf, osb), (dbuf, odb)):
                        v = inb[pl.ds(i, 16)]
                        jj = ((v >= n4).astype(jnp.int32)
                              + (v >= 2 * n4).astype(jnp.int32)
                              + (v >= 3 * n4).astype(jnp.int32))
                        tv = 4 * v - jj * (4 * n4 - 1)
                        plsc.store_scatter(outb, [pi], tv)

                pltpu.sync_copy(onesv, acc.at[odb], add=True)
                pltpu.sync_copy(osb, srcp_h.at[pl.ds(k_ * ch, ch)])
                pltpu.sync_copy(odb, dstp_h.at[pl.ds(k_ * ch, ch)])
        plsc.subcore_barrier()
        for j in range((nz + _NS - 1) // _NS):
            kk = tt + _NS * j

            @pl.when(jnp.logical_and(kk < nz, cc == 0))
            def _():
                pltpu.sync_copy(acc.at[pl.ds(kk * zch, zch)], stage)
                pltpu.sync_copy(stage, d0_h.at[pl.ds(kk * zch, zch)])

            @pl.when(jnp.logical_and(kk < nz, cc == 1))
            def _():
                pltpu.sync_copy(acc.at[pl.ds(kk * zch, zch)], stage)
                pltpu.sync_copy(stage, d1_h.at[pl.ds(kk * zch, zch)])

    return k(ei1, pattern_hbm, ones_hbm, zeros_hbm)


def _sc_scatter(m, dst, zrows_hbm, n):
    e = dst.shape[0]
    hc = m.shape[2]
    ch = 400
    per_t = e // _NS
    assert per_t % ch == 0
    ng = per_t // ch
    zch = zrows_hbm.shape[0]
    nz = n // zch
    assert nz * zch == n

    @functools.partial(
        pl.kernel,
        out_type=jax.ShapeDtypeStruct((_NC, n, hc), jnp.float32),
        mesh=_mesh(),
        compiler_params=pltpu.CompilerParams(use_tc_tiling_on_sc=False),
        scratch_types=[
            pltpu.VMEM_SHARED((n, hc), jnp.float32),
            pltpu.VMEM((ch,), jnp.int32),
            pltpu.VMEM((ch,), jnp.int32),
            pltpu.VMEM((ch, hc), jnp.float32),
            pltpu.VMEM((ch, hc), jnp.float32),
            pltpu.SemaphoreType.DMA, pltpu.SemaphoreType.DMA,
            pltpu.SemaphoreType.DMA, pltpu.SemaphoreType.DMA,
        ],
    )
    def k(m_hbm, dst_hbm, zrows_h, out_hbm, acc, ib0, ib1, ub0, ub1,
          is0, is1, us0, us1):
        cc = lax.axis_index("c")
        tt = lax.axis_index("s")
        pltpu.sync_copy(zrows_h, ub0)
        for j in range((nz + _NS - 1) // _NS):
            kk = tt + _NS * j

            @pl.when(kk < nz)
            def _():
                pltpu.sync_copy(ub0, acc.at[pl.ds(kk * zch, zch)])
        plsc.subcore_barrier()

        base = tt * per_t
        ib = (ib0, ib1)
        ub = (ub0, ub1)
        isem = (is0, is1)
        usem = (us0, us1)

        def load(g, b):
            di = pltpu.async_copy(
                dst_hbm.at[pl.ds(base + g * ch, ch)], ib[b], isem[b])
            du = pltpu.async_copy(
                m_hbm.at[cc, pl.ds(base + g * ch, ch)], ub[b], usem[b])
            return di, du

        pend = {0: load(0, 0)}
        if ng > 1:
            pend[1] = load(1, 1)
        for g in range(ng):
            b = g & 1
            pend[g][0].wait()
            pend[g][1].wait()
            pltpu.sync_copy(ub[b], acc.at[ib[b]], add=True)
            if g + 2 < ng:
                pend[g + 2] = load(g + 2, b)
        plsc.subcore_barrier()
        for j in range((nz + _NS - 1) // _NS):
            kk = tt + _NS * j

            @pl.when(kk < nz)
            def _():
                pltpu.sync_copy(acc.at[pl.ds(kk * zch, zch)], ub0)
                pltpu.sync_copy(ub0, out_hbm.at[cc, pl.ds(kk * zch, zch)])

    return k(m, dst, zrows_hbm)


def _dot(a, b):
    return jax.lax.dot_general(a, b, (((1,), (0,)), ((), ())),
                               preferred_element_type=jnp.float32)


def _dot0(a, b):
    return jax.lax.dot_general(a, b, (((0,), (0,)), ((), ())),
                               preferred_element_type=jnp.float32)


def _pack(x):
    r, c = x.shape
    k = 128 // c
    x3 = x.reshape(r // k, k, c)
    return jnp.concatenate([x3[:, j] for j in range(k)], axis=1)


def _unpack(p, c):
    q = p.shape[0]
    k = 128 // c
    g3 = jnp.stack([p[:, j * c:(j + 1) * c] for j in range(k)], axis=1)
    return g3.reshape(q * k, c)


def _full(shape):
    return pl.BlockSpec(shape, lambda i: tuple(0 for _ in shape))


def _prep1(xTr, tw1bd, tb1t, tw2bd, tb2t, w1abd, b1t, bnq):
    n4 = xTr.shape[1]

    def body(x_ref, tw1, tb1r, tw2, tb2r, wa, b1r, x0_ref, g1_ref):
        x1 = jnp.maximum(_dot0(x_ref[...], tw1[...]) + tb1r[...], 0.0)
        x0p = jnp.maximum(_dot(x1, tw2[...]) + tb2r[...], 0.0)
        x0_ref[...] = x0p
        g1_ref[...] = _dot(x0p, wa[...]) + b1r[...]

    return pl.pallas_call(
        body,
        grid=(pl.cdiv(n4, bnq),),
        in_specs=[
            pl.BlockSpec((xTr.shape[0], bnq), lambda i: (0, i)),
            _full(tw1bd.shape), _full(tb1t.shape), _full(tw2bd.shape),
            _full(tb2t.shape), _full(w1abd.shape), _full(b1t.shape),
        ],
        out_specs=[
            pl.BlockSpec((bnq, 128), lambda i: (i, 0)),
            pl.BlockSpec((bnq, 128), lambda i: (i, 0)),
        ],
        out_shape=[
            jax.ShapeDtypeStruct((n4, 128), jnp.float32),
            jax.ShapeDtypeStruct((n4, 128), jnp.float32),
        ],
    )(xTr, tw1bd, tb1t, tw2bd, tb2t, w1abd, b1t)


def _edge_mlp(garrp, eaR, w1big, w2lo, w2hi, b2lo, b2hi, aub):
    e4, gw = garrp.shape

    def body(g_ref, ea_ref, w1, wlo, whi, blo, bhi, out_ref):
        et = _dot0(ea_ref[...], w1[...])
        h = jnp.maximum(g_ref[...] + et, 0.0)
        out_ref[0] = jnp.maximum(_dot(h, wlo[...]) + blo[...], 0.0)
        out_ref[1] = jnp.maximum(_dot(h, whi[...]) + bhi[...], 0.0)

    return pl.pallas_call(
        body,
        grid=(pl.cdiv(e4, aub),),
        in_specs=[
            pl.BlockSpec((aub, gw), lambda i: (i, 0)),
            pl.BlockSpec((64, aub), lambda i: (0, i)),
            _full(w1big.shape), _full(w2lo.shape), _full(w2hi.shape),
            _full(b2lo.shape), _full(b2hi.shape),
        ],
        out_specs=pl.BlockSpec((2, aub, 128), lambda i: (0, i, 0)),
        out_shape=jax.ShapeDtypeStruct((2, e4, 128), jnp.float32),
    )(garrp, eaR, w1big, w2lo, w2hi, b2lo, b2hi)


def _edge_mlp2(glo, ghi, eaR, w1lo, w1hi, w2ll, w2hl, w2lh, w2hh,
               b2lo, b2hi, aub):
    e4 = glo.shape[0]

    def body(gl_ref, gh_ref, ea_ref, wlo, whi, ll, hl, lh, hh, blo, bhi,
             out_ref):
        ea = ea_ref[...]
        h_lo = jnp.maximum(gl_ref[...] + _dot0(ea, wlo[...]), 0.0)
        h_hi = jnp.maximum(gh_ref[...] + _dot0(ea, whi[...]), 0.0)
        out_ref[0] = jnp.maximum(
            _dot(h_lo, ll[...]) + _dot(h_hi, hl[...]) + blo[...], 0.0)
        out_ref[1] = jnp.maximum(
            _dot(h_lo, lh[...]) + _dot(h_hi, hh[...]) + bhi[...], 0.0)

    return pl.pallas_call(
        body,
        grid=(pl.cdiv(e4, aub),),
        in_specs=[
            pl.BlockSpec((aub, 128), lambda i: (i, 0)),
            pl.BlockSpec((aub, 128), lambda i: (i, 0)),
            pl.BlockSpec((64, aub), lambda i: (0, i)),
            _full(w1lo.shape), _full(w1hi.shape), _full(w2ll.shape),
            _full(w2hl.shape), _full(w2lh.shape), _full(w2hh.shape),
            _full(b2lo.shape), _full(b2hi.shape),
        ],
        out_specs=pl.BlockSpec((2, aub, 128), lambda i: (0, i, 0)),
        out_shape=jax.ShapeDtypeStruct((2, e4, 128), jnp.float32),
    )(glo, ghi, eaR, w1lo, w1hi, w2ll, w2hl, w2lh, w2hh, b2lo, b2hi)


def _mid(aggp, d4, x0p, bdlos, bdhis, bdxs, b1ts, m128, bnq):
    n4 = x0p.shape[0]

    def body(a_ref, d_ref, x0_ref, wl0, wl1, wh0, wh1, wx0, wx1, b0, b1r,
             mr, lo_ref, hi_ref):
        inv4 = 1.0 / jnp.maximum(d_ref[...], 1.0)
        invp = _dot(inv4, mr[...])
        a = a_ref[...]
        lo_ref[...] = (invp * (_dot(a[0], wl0[...]) + _dot(a[1], wh0[...]))
                       + _dot(x0_ref[...], wx0[...]) + b0[...])
        hi_ref[...] = (invp * (_dot(a[0], wl1[...]) + _dot(a[1], wh1[...]))
                       + _dot(x0_ref[...], wx1[...]) + b1r[...])

    return pl.pallas_call(
        body,
        grid=(pl.cdiv(n4, bnq),),
        in_specs=[
            pl.BlockSpec((2, bnq, 128), lambda i: (0, i, 0)),
            pl.BlockSpec((bnq, 4), lambda i: (i, 0)),
            pl.BlockSpec((bnq, 128), lambda i: (i, 0)),
            _full(bdlos[0].shape), _full(bdlos[1].shape),
            _full(bdhis[0].shape), _full(bdhis[1].shape),
            _full(bdxs[0].shape), _full(bdxs[1].shape),
            _full(b1ts[0].shape), _full(b1ts[1].shape),
            _full(m128.shape),
        ],
        out_specs=[
            pl.BlockSpec((bnq, 128), lambda i: (i, 0)),
            pl.BlockSpec((bnq, 128), lambda i: (i, 0)),
        ],
        out_shape=[
            jax.ShapeDtypeStruct((n4, 128), jnp.float32),
            jax.ShapeDtypeStruct((n4, 128), jnp.float32),
        ],
    )(aggp, d4, x0p, bdlos[0], bdlos[1], bdhis[0], bdhis[1],
      bdxs[0], bdxs[1], b1ts[0], b1ts[1], m128)


def _head(aggp, d4, x0p, bdlo, bdhi, bdx, pb1t, bdw2, pb2t, m64, bnq):
    n4 = x0p.shape[0]

    def body(a_ref, d_ref, x0_ref, wl, wh, wxr, b1r, w2r, b2r, mr, out_ref):
        inv4 = 1.0 / jnp.maximum(d_ref[...], 1.0)
        invp = _dot(inv4, mr[...])
        a = a_ref[...]
        h = jnp.maximum(
            invp * (_dot(a[0], wl[...]) + _dot(a[1], wh[...]))
            + _dot(x0_ref[...], wxr[...]) + b1r[...], 0.0)
        out_ref[...] = _dot(h, w2r[...]) + b2r[...]

    return pl.pallas_call(
        body,
        grid=(pl.cdiv(n4, bnq),),
        in_specs=[
            pl.BlockSpec((2, bnq, 128), lambda i: (0, i, 0)),
            pl.BlockSpec((bnq, 4), lambda i: (i, 0)),
            pl.BlockSpec((bnq, 128), lambda i: (i, 0)),
            _full(bdlo.shape), _full(bdhi.shape), _full(bdx.shape),
            _full(pb1t.shape), _full(bdw2.shape), _full(pb2t.shape),
            _full(m64.shape),
        ],
        out_specs=pl.BlockSpec((bnq, 4), lambda i: (i, 0)),
        out_shape=jax.ShapeDtypeStruct((n4, 4), jnp.float32),
    )(aggp, d4, x0p, bdlo, bdhi, bdx, pb1t, bdw2, pb2t, m64)


def kernel(x, sx, edge_index, edge_attr, batch, tW1, tb1, tW2, tb2,
           s1W1, s1b1, s1W2, s1b2, s2W1, s2b1, s2W2, s2b2,
           pW1, pb1, pW2, pb2):
    n = x.shape[0]
    e = edge_attr.shape[0]
    n4 = n // 4
    e4 = e // 4
    bnq = 512
    aub = 4096
    pch = 4000

    z1 = jnp.zeros((2000,), jnp.float32)
    z32 = jnp.zeros((400, 32), jnp.float32)
    onesp = jnp.ones((pch,), jnp.float32)
    ar = jnp.arange(pch, dtype=jnp.int32)
    pattern = 4 * (ar % (pch // 4)) + ar // (pch // 4)

    i4 = jnp.eye(4, dtype=jnp.float32)

    def bdl(w):
        return (w[:, None, None, :] * i4[None, :, :, None]).reshape(
            4 * w.shape[0], 4 * w.shape[1])

    def bdr(w):
        return (i4[:, None, :, None] * w[None, :, None, :]).reshape(
            4 * w.shape[0], 4 * w.shape[1])

    def t4(b):
        return jnp.tile(b, 4).reshape(1, -1)

    m128 = bdr(jnp.ones((1, 32), jnp.float32))
    m64 = bdr(jnp.ones((1, 16), jnp.float32))

    srcp, dstp, degp0, degp1 = _sc_index_prep(edge_index.reshape(2 * e),
                                              pattern, onesp, z1, n)
    d4 = (degp0 + degp1).reshape(n4, 4)

    eaR = edge_attr.T.reshape(64, e4)
    xTr = x.T.reshape(448, n4)

    x0p, g1p = _prep1(xTr, bdl(tW1), t4(tb1), bdr(tW2), t4(tb2),
                      bdr(s1W1[:32]), t4(s1b1), bnq)

    garr1 = _sc_gather(g1p.reshape(n, 32), srcp, 1000)
    m1 = _edge_mlp(garr1.reshape(e4, 128), eaR, bdl(s1W1[32:48]),
                   bdr(s1W2[:, :32]), bdr(s1W2[:, 32:]),
                   t4(s1b2[:32]), t4(s1b2[32:]), aub)
    agg1 = _sc_scatter(m1.reshape(2, e, 32), dstp, z32, n)

    g2lo, g2hi = _mid(
        agg1.reshape(2, n4, 128), d4, x0p,
        (bdr(s2W1[:32, :32]), bdr(s2W1[:32, 32:])),
        (bdr(s2W1[32:64, :32]), bdr(s2W1[32:64, 32:])),
        (bdr(s2W1[64:96, :32]), bdr(s2W1[64:96, 32:])),
        (t4(s2b1[:32]), t4(s2b1[32:])), m128, bnq)

    glo = _sc_gather(g2lo.reshape(n, 32), srcp, 1000)
    ghi = _sc_gather(g2hi.reshape(n, 32), srcp, 1000)
    m2 = _edge_mlp2(glo.reshape(e4, 128), ghi.reshape(e4, 128), eaR,
                    bdl(s2W1[96:112, :32]), bdl(s2W1[96:112, 32:]),
                    bdr(s2W2[:32, :32]), bdr(s2W2[32:, :32]),
                    bdr(s2W2[:32, 32:]), bdr(s2W2[32:, 32:]),
                    t4(s2b2[:32]), t4(s2b2[32:]), aub)
    agg2 = _sc_scatter(m2.reshape(2, e, 32), dstp, z32, n)

    y4 = _head(agg2.reshape(2, n4, 128), d4, x0p, bdr(pW1[:32]),
               bdr(pW1[32:64]), bdr(pW1[64:96]), t4(pb1), bdr(pW2),
               t4(pb2), m64, bnq)
    return y4.T.reshape(n, 1)
